# baseline (speedup 1.0000x reference)
import jax
import jax.numpy as jnp
from jax import lax
from jax.experimental import pallas as pl
from jax.experimental.pallas import tpu as pltpu
from jax.experimental.pallas import tpu_sc as plsc

N = 10000
E = 320000
F_IN = 128
H = 128
C = 40

NC = 2
NS = 16
NW = NC * NS
EPW = E // NW
K = 128
EPW_PAD = 10240
NCHUNK = EPW_PAD // K
PAD_DST = N
NPAD = 10240
RPT = NPAD // NS
ZR = RPT // 5
NP_CNT = 10240
CPT = NP_CNT // NS


def _make_seg_sum(D, with_count):
    mesh = plsc.VectorSubcoreMesh(
        core_axis_name="c", subcore_axis_name="s",
        num_cores=NC, num_subcores=NS)
    out_type = [jax.ShapeDtypeStruct((NC, NPAD, D), jnp.float32)]
    scratch = [
        pltpu.VMEM_SHARED((NPAD, D), jnp.float32),
        pltpu.VMEM((NCHUNK, K), jnp.int32),
        pltpu.VMEM((NCHUNK, K), jnp.int32),
        pltpu.VMEM((K, D), jnp.float32),
        pltpu.VMEM((K, D), jnp.float32),
        pltpu.VMEM((ZR, D), jnp.float32),
        pltpu.SemaphoreType.DMA,
        pltpu.SemaphoreType.DMA,
    ]
    if with_count:
        out_type.append(jax.ShapeDtypeStruct((NC, NP_CNT), jnp.float32))
        scratch += [
            pltpu.VMEM_SHARED((NP_CNT,), jnp.float32),
            pltpu.VMEM((CPT,), jnp.float32),
            pltpu.VMEM((K,), jnp.float32),
        ]

    def body(table, srcw, dstw, *refs):
        if with_count:
            (out, cnt_out, acc, src_v, dst_v, rows_a, rows_b, zbuf,
             sem_a, sem_b, cnt_acc, zc, ones_v) = refs
        else:
            (out, acc, src_v, dst_v, rows_a, rows_b, zbuf,
             sem_a, sem_b) = refs
        c = lax.axis_index("c")
        s = lax.axis_index("s")
        wid = c * NS + s

        cpl = D // 16

        def zfill(k, _):
            zbuf[k // cpl, pl.ds((k % cpl) * 16, 16)] = jnp.zeros(
                (16,), jnp.float32)
            return 0

        lax.fori_loop(0, ZR * cpl, zfill, 0)
        for t in range(5):
            pltpu.sync_copy(zbuf, acc.at[pl.ds(s * RPT + t * ZR, ZR)])
        if with_count:
            def zc_fill(k, _):
                zc[pl.ds(k * 16, 16)] = jnp.zeros((16,), jnp.float32)
                return 0

            lax.fori_loop(0, CPT // 16, zc_fill, 0)
            pltpu.sync_copy(zc, cnt_acc.at[pl.ds(s * CPT, CPT)])

            def ones_fill(k, _):
                ones_v[pl.ds(k * 16, 16)] = jnp.ones((16,), jnp.float32)
                return 0

            lax.fori_loop(0, K // 16, ones_fill, 0)

        pltpu.sync_copy(srcw.at[wid], src_v)
        pltpu.sync_copy(dstw.at[wid], dst_v)
        plsc.subcore_barrier()

        def chunk(j, _):
            pltpu.async_copy(table.at[src_v.at[j]], rows_a, sem_a).wait()
            pltpu.sync_copy(rows_a, acc.at[dst_v.at[j]], add=True)
            if with_count:
                pltpu.sync_copy(ones_v, cnt_acc.at[dst_v.at[j]], add=True)
            return 0

        lax.fori_loop(0, NCHUNK, chunk, 0)
        plsc.subcore_barrier()

        pltpu.sync_copy(acc.at[pl.ds(s * RPT, RPT)],
                        out.at[c, pl.ds(s * RPT, RPT)])
        if with_count:
            pltpu.sync_copy(cnt_acc.at[pl.ds(s * CPT, CPT)],
                            cnt_out.at[c, pl.ds(s * CPT, CPT)])

    return pl.kernel(body, out_type=out_type, mesh=mesh,
                     scratch_types=scratch,
                     compiler_params=pltpu.CompilerParams(
                         use_tc_tiling_on_sc=False),
                     name=f"sc_seg_sum_d{D}" + ("_cnt" if with_count else ""))


HD = F_IN // 2
_seg_sum_cnt_64 = _make_seg_sum(HD, True)
_seg_sum_64 = _make_seg_sum(HD, False)
_seg_sum_40 = _make_seg_sum(C, False)


BR = 1000


def _tc_a_body(aggpa, aggpb, cntp, x, w1l, w1r, b1, w2l, w2r, b2,
               p2_o, r2b_o, invc_o, dinv_o):
    agg = jnp.concatenate([aggpa[0] + aggpa[1], aggpb[0] + aggpb[1]], axis=1)
    cnt = cntp[0, 0] + cntp[0, 1]
    invc = (1.0 / jnp.maximum(cnt, 1.0))[:, None]
    dinv = lax.rsqrt(cnt + 1.0)[:, None]
    mean = agg * invc
    h1 = jnp.dot(mean, w1l[...], preferred_element_type=jnp.float32)
    h1 = h1 + jnp.dot(x[...], w1r[...], preferred_element_type=jnp.float32)
    h1 = jnp.maximum(h1 + b1[...], 0.0)
    p2_o[...] = jnp.dot(h1, w2l[...], preferred_element_type=jnp.float32)
    r2b_o[...] = jnp.dot(h1, w2r[...],
                         preferred_element_type=jnp.float32) + b2[...]
    invc_o[...] = invc
    dinv_o[...] = dinv


def _tc_b_body(aggp, invc, r2b, wc, dinv, y_o):
    h2 = (aggp[0] + aggp[1]) * invc[...] + r2b[...]
    xw = jnp.dot(h2, wc[...], preferred_element_type=jnp.float32)
    y_o[...] = xw * dinv[...]


def _tc_c_body(aggp, y, dinv, bc, out_o):
    out_o[...] = dinv[...] * (aggp[0] + aggp[1] + y[...]) + bc[...]


def _row_spec(d):
    return pl.BlockSpec((BR, d), lambda i: (i, 0))


def _part_spec(d):
    return pl.BlockSpec((NC, BR, d), lambda i: (0, i, 0))


def _full_spec(a, b):
    return pl.BlockSpec((a, b), lambda i: (0, 0))


_tc_a = pl.pallas_call(
    _tc_a_body,
    grid=(N // BR,),
    in_specs=[
        _part_spec(HD),
        _part_spec(HD),
        pl.BlockSpec((1, NC, BR), lambda i: (i, 0, 0)),
        _row_spec(F_IN),
        _full_spec(F_IN, H), _full_spec(F_IN, H), _full_spec(1, H),
        _full_spec(H, C), _full_spec(H, C), _full_spec(1, C),
    ],
    out_specs=[_row_spec(C), _row_spec(C), _row_spec(1), _row_spec(1)],
    out_shape=[
        jax.ShapeDtypeStruct((N, C), jnp.float32),
        jax.ShapeDtypeStruct((N, C), jnp.float32),
        jax.ShapeDtypeStruct((N, 1), jnp.float32),
        jax.ShapeDtypeStruct((N, 1), jnp.float32),
    ],
    name="tc_dense_a",
)

_tc_b = pl.pallas_call(
    _tc_b_body,
    grid=(N // BR,),
    in_specs=[_part_spec(C), _row_spec(1), _row_spec(C),
              _full_spec(C, C), _row_spec(1)],
    out_specs=[_row_spec(C)],
    out_shape=[jax.ShapeDtypeStruct((N, C), jnp.float32)],
    name="tc_dense_b",
)

_tc_c = pl.pallas_call(
    _tc_c_body,
    grid=(N // BR,),
    in_specs=[_part_spec(C), _row_spec(C), _row_spec(1), _full_spec(1, C)],
    out_specs=[_row_spec(C)],
    out_shape=[jax.ShapeDtypeStruct((N, C), jnp.float32)],
    name="tc_dense_c",
)


@jax.jit
def kernel(x, edge_index, W1l, W1r, b1, W2l, W2r, b2, Wc, bc):
    src2 = edge_index[0].reshape(NW, EPW)
    dst2 = edge_index[1].reshape(NW, EPW)
    npad = EPW_PAD - EPW
    src3 = jnp.concatenate(
        [src2, jnp.zeros((NW, npad), jnp.int32)], axis=1
    ).reshape(NW, NCHUNK, K)
    dst3 = jnp.concatenate(
        [dst2, jnp.full((NW, npad), PAD_DST, jnp.int32)], axis=1
    ).reshape(NW, NCHUNK, K)
    b1r = b1.reshape(1, H)
    b2r = b2.reshape(1, C)
    bcr = bc.reshape(1, C)

    xa = x[:, :HD]
    xb = x[:, HD:]
    agg1pa, cntp_pad = _seg_sum_cnt_64(xa, src3, dst3)
    (agg1pb,) = _seg_sum_64(xb, src3, dst3)
    cntp = cntp_pad[:, :N].reshape(NC, N // BR, BR).transpose(1, 0, 2)
    p2, r2b, invc, dinv = _tc_a(agg1pa, agg1pb, cntp, x, W1l, W1r, b1r,
                                W2l, W2r, b2r)
    (agg2p,) = _seg_sum_40(p2, src3, dst3)
    (y,) = _tc_b(agg2p, invc, r2b, Wc, dinv)
    (agg3p,) = _seg_sum_40(y, src3, dst3)
    (out,) = _tc_c(agg3p, y, dinv, bcr)
    return out


# speedup vs baseline: 1.0001x; 1.0001x over previous
import jax
import jax.numpy as jnp
from jax import lax
from jax.experimental import pallas as pl
from jax.experimental.pallas import tpu as pltpu
from jax.experimental.pallas import tpu_sc as plsc

N = 10000
E = 320000
F_IN = 128
H = 128
C = 40

NC = 2
NS = 16
NW = NC * NS
EPW = E // NW
K = 128
EPW_PAD = 10240
NCHUNK = EPW_PAD // K
PAD_DST = N
NPAD = 10240
RPT = NPAD // NS
ZR = RPT // 5
NP_CNT = 10240
CPT = NP_CNT // NS


def _make_seg_sum(D, with_count):
    mesh = plsc.VectorSubcoreMesh(
        core_axis_name="c", subcore_axis_name="s",
        num_cores=NC, num_subcores=NS)
    out_type = [jax.ShapeDtypeStruct((NC, NPAD, D), jnp.float32)]
    scratch = [
        pltpu.VMEM_SHARED((NPAD, D), jnp.float32),
        pltpu.VMEM((NCHUNK, K), jnp.int32),
        pltpu.VMEM((NCHUNK, K), jnp.int32),
        pltpu.VMEM((K, D), jnp.float32),
        pltpu.VMEM((K, D), jnp.float32),
        pltpu.VMEM((ZR, D), jnp.float32),
        pltpu.SemaphoreType.DMA,
        pltpu.SemaphoreType.DMA,
    ]
    if with_count:
        out_type.append(jax.ShapeDtypeStruct((NC, NP_CNT), jnp.float32))
        scratch += [
            pltpu.VMEM_SHARED((NP_CNT,), jnp.float32),
            pltpu.VMEM((CPT,), jnp.float32),
            pltpu.VMEM((K,), jnp.float32),
        ]

    def body(table, srcw, dstw, *refs):
        if with_count:
            (out, cnt_out, acc, src_v, dst_v, rows_a, rows_b, zbuf,
             sem_a, sem_b, cnt_acc, zc, ones_v) = refs
        else:
            (out, acc, src_v, dst_v, rows_a, rows_b, zbuf,
             sem_a, sem_b) = refs
        c = lax.axis_index("c")
        s = lax.axis_index("s")
        wid = c * NS + s

        cpl = D // 16

        def zfill(k, _):
            zbuf[k // cpl, pl.ds((k % cpl) * 16, 16)] = jnp.zeros(
                (16,), jnp.float32)
            return 0

        lax.fori_loop(0, ZR * cpl, zfill, 0)
        for t in range(5):
            pltpu.sync_copy(zbuf, acc.at[pl.ds(s * RPT + t * ZR, ZR)])
        if with_count:
            def zc_fill(k, _):
                zc[pl.ds(k * 16, 16)] = jnp.zeros((16,), jnp.float32)
                return 0

            lax.fori_loop(0, CPT // 16, zc_fill, 0)
            pltpu.sync_copy(zc, cnt_acc.at[pl.ds(s * CPT, CPT)])

            def ones_fill(k, _):
                ones_v[pl.ds(k * 16, 16)] = jnp.ones((16,), jnp.float32)
                return 0

            lax.fori_loop(0, K // 16, ones_fill, 0)

        pltpu.sync_copy(srcw.at[wid], src_v)
        pltpu.sync_copy(dstw.at[wid], dst_v)
        plsc.subcore_barrier()

        def chunk(j, _):
            pltpu.async_copy(table.at[src_v.at[j]], rows_a, sem_a).wait()
            pltpu.sync_copy(rows_a, acc.at[dst_v.at[j]], add=True)
            if with_count:
                pltpu.sync_copy(ones_v, cnt_acc.at[dst_v.at[j]], add=True)
            return 0

        lax.fori_loop(0, NCHUNK, chunk, 0)
        plsc.subcore_barrier()

        pltpu.sync_copy(acc.at[pl.ds(s * RPT, RPT)],
                        out.at[c, pl.ds(s * RPT, RPT)])
        if with_count:
            pltpu.sync_copy(cnt_acc.at[pl.ds(s * CPT, CPT)],
                            cnt_out.at[c, pl.ds(s * CPT, CPT)])

    return pl.kernel(body, out_type=out_type, mesh=mesh,
                     scratch_types=scratch,
                     compiler_params=pltpu.CompilerParams(
                         use_tc_tiling_on_sc=False),
                     name=f"sc_seg_sum_d{D}" + ("_cnt" if with_count else ""))


HD = F_IN // 2
_seg_sum_cnt_64 = _make_seg_sum(HD, True)
_seg_sum_64 = _make_seg_sum(HD, False)
_seg_sum_40 = _make_seg_sum(C, False)


BR = 1000


def _tc_a_body(aggpa, aggpb, cntp, x, w1l, w1r, b1, w2l, w2r, b2,
               p2_o, r2b_o, invc_o, dinv_o):
    agg = jnp.concatenate([aggpa[0] + aggpa[1], aggpb[0] + aggpb[1]], axis=1)
    cnt = cntp[0, 0] + cntp[0, 1]
    invc = (1.0 / jnp.maximum(cnt, 1.0))[:, None]
    dinv = lax.rsqrt(cnt + 1.0)[:, None]
    mean = agg * invc
    h1 = jnp.dot(mean, w1l[...], preferred_element_type=jnp.float32)
    h1 = h1 + jnp.dot(x[...], w1r[...], preferred_element_type=jnp.float32)
    h1 = jnp.maximum(h1 + b1[...], 0.0)
    p2_o[...] = jnp.dot(h1, w2l[...], preferred_element_type=jnp.float32)
    r2b_o[...] = jnp.dot(h1, w2r[...],
                         preferred_element_type=jnp.float32) + b2[...]
    invc_o[...] = invc
    dinv_o[...] = dinv


def _tc_b_body(aggp, invc, r2b, wc, dinv, y_o):
    h2 = (aggp[0] + aggp[1]) * invc[...] + r2b[...]
    xw = jnp.dot(h2, wc[...], preferred_element_type=jnp.float32)
    y_o[...] = xw * dinv[...]


def _tc_c_body(aggp, y, dinv, bc, out_o):
    out_o[...] = dinv[...] * (aggp[0] + aggp[1] + y[...]) + bc[...]


def _row_spec(d):
    return pl.BlockSpec((BR, d), lambda i: (i, 0))


def _part_spec(d):
    return pl.BlockSpec((NC, BR, d), lambda i: (0, i, 0))


def _full_spec(a, b):
    return pl.BlockSpec((a, b), lambda i: (0, 0))


_tc_a = pl.pallas_call(
    _tc_a_body,
    grid=(N // BR,),
    in_specs=[
        _part_spec(HD),
        _part_spec(HD),
        pl.BlockSpec((1, NC, BR), lambda i: (i, 0, 0)),
        _row_spec(F_IN),
        _full_spec(F_IN, H), _full_spec(F_IN, H), _full_spec(1, H),
        _full_spec(H, C), _full_spec(H, C), _full_spec(1, C),
    ],
    out_specs=[_row_spec(C), _row_spec(C), _row_spec(1), _row_spec(1)],
    out_shape=[
        jax.ShapeDtypeStruct((N, C), jnp.float32),
        jax.ShapeDtypeStruct((N, C), jnp.float32),
        jax.ShapeDtypeStruct((N, 1), jnp.float32),
        jax.ShapeDtypeStruct((N, 1), jnp.float32),
    ],
    name="tc_dense_a",
)

_tc_b = pl.pallas_call(
    _tc_b_body,
    grid=(N // BR,),
    in_specs=[_part_spec(C), _row_spec(1), _row_spec(C),
              _full_spec(C, C), _row_spec(1)],
    out_specs=[_row_spec(C)],
    out_shape=[jax.ShapeDtypeStruct((N, C), jnp.float32)],
    name="tc_dense_b",
)

_tc_c = pl.pallas_call(
    _tc_c_body,
    grid=(N // BR,),
    in_specs=[_part_spec(C), _row_spec(C), _row_spec(1), _full_spec(1, C)],
    out_specs=[_row_spec(C)],
    out_shape=[jax.ShapeDtypeStruct((N, C), jnp.float32)],
    name="tc_dense_c",
)


@jax.jit
def kernel(x, edge_index, W1l, W1r, b1, W2l, W2r, b2, Wc, bc):
    src2 = edge_index[0].reshape(NW, EPW)
    dst2 = edge_index[1].reshape(NW, EPW)
    npad = EPW_PAD - EPW
    src3 = jnp.concatenate(
        [src2, jnp.zeros((NW, npad), jnp.int32)], axis=1
    ).reshape(NW, NCHUNK, K)
    pad_rows = PAD_DST + jnp.arange(npad, dtype=jnp.int32)
    dst3 = jnp.concatenate(
        [dst2, jnp.broadcast_to(pad_rows, (NW, npad))], axis=1
    ).reshape(NW, NCHUNK, K)
    b1r = b1.reshape(1, H)
    b2r = b2.reshape(1, C)
    bcr = bc.reshape(1, C)

    xa = x[:, :HD]
    xb = x[:, HD:]
    agg1pa, cntp_pad = _seg_sum_cnt_64(xa, src3, dst3)
    (agg1pb,) = _seg_sum_64(xb, src3, dst3)
    cntp = cntp_pad[:, :N].reshape(NC, N // BR, BR).transpose(1, 0, 2)
    p2, r2b, invc, dinv = _tc_a(agg1pa, agg1pb, cntp, x, W1l, W1r, b1r,
                                W2l, W2r, b2r)
    (agg2p,) = _seg_sum_40(p2, src3, dst3)
    (y,) = _tc_b(agg2p, invc, r2b, Wc, dinv)
    (agg3p,) = _seg_sum_40(y, src3, dst3)
    (out,) = _tc_c(agg3p, y, dinv, bcr)
    return out


# speedup vs baseline: 2.3051x; 2.3050x over previous
import jax
import jax.numpy as jnp
from jax import lax
from jax.experimental import pallas as pl
from jax.experimental.pallas import tpu as pltpu
from jax.experimental.pallas import tpu_sc as plsc

N = 10000
E = 320000
F_IN = 128
H = 128
C = 40

NC = 2
NS = 16
NW = NC * NS
EPW = E // NW
K = 80
NCHUNK = EPW // K
G = 5
NB = NCHUNK // G
NPAD = 10240
RPT = NPAD // NS
ZR = RPT // 5
NP_CNT = 10240
CPT = NP_CNT // NS


def _make_seg_sum(D, with_count):
    mesh = plsc.VectorSubcoreMesh(
        core_axis_name="c", subcore_axis_name="s",
        num_cores=NC, num_subcores=NS)
    out_type = [jax.ShapeDtypeStruct((NC, NPAD, D), jnp.float32)]
    scratch = [
        pltpu.VMEM_SHARED((NPAD, D), jnp.float32),
        pltpu.VMEM((NCHUNK, K), jnp.int32),
        pltpu.VMEM((NCHUNK, K), jnp.int32),
        pltpu.VMEM((G, K, D), jnp.float32),
        pltpu.VMEM((ZR, D), jnp.float32),
        pltpu.SemaphoreType.DMA,
    ]
    if with_count:
        out_type.append(jax.ShapeDtypeStruct((NC, NP_CNT), jnp.float32))
        scratch += [
            pltpu.VMEM_SHARED((NP_CNT,), jnp.float32),
            pltpu.VMEM((CPT,), jnp.float32),
            pltpu.VMEM((K,), jnp.float32),
        ]

    def body(table, srcw, dstw, *refs):
        if with_count:
            (out, cnt_out, acc, src_v, dst_v, rows_v, zbuf, sem,
             cnt_acc, zc, ones_v) = refs
        else:
            out, acc, src_v, dst_v, rows_v, zbuf, sem = refs
        c = lax.axis_index("c")
        s = lax.axis_index("s")
        wid = c * NS + s

        cpl = D // 16

        def zfill(k, _):
            zbuf[k // cpl, pl.ds((k % cpl) * 16, 16)] = jnp.zeros(
                (16,), jnp.float32)
            return 0

        lax.fori_loop(0, ZR * cpl, zfill, 0)
        for t in range(5):
            pltpu.sync_copy(zbuf, acc.at[pl.ds(s * RPT + t * ZR, ZR)])
        if with_count:
            def zc_fill(k, _):
                zc[pl.ds(k * 16, 16)] = jnp.zeros((16,), jnp.float32)
                return 0

            lax.fori_loop(0, CPT // 16, zc_fill, 0)
            pltpu.sync_copy(zc, cnt_acc.at[pl.ds(s * CPT, CPT)])

            def ones_fill(k, _):
                ones_v[pl.ds(k * 16, 16)] = jnp.ones((16,), jnp.float32)
                return 0

            lax.fori_loop(0, K // 16, ones_fill, 0)

        pltpu.sync_copy(srcw.at[wid], src_v)
        pltpu.sync_copy(dstw.at[wid], dst_v)
        plsc.subcore_barrier()

        def batch(t, _):
            base = t * G
            for g in range(G):
                pltpu.async_copy(table.at[src_v.at[base + g]],
                                 rows_v.at[g], sem)
            for g in range(G):
                pltpu.make_async_copy(table.at[src_v.at[base + g]],
                                      rows_v.at[g], sem).wait()
            for g in range(G):
                pltpu.sync_copy(rows_v.at[g], acc.at[dst_v.at[base + g]],
                                add=True)
                if with_count:
                    pltpu.sync_copy(ones_v, cnt_acc.at[dst_v.at[base + g]],
                                    add=True)
            return 0

        lax.fori_loop(0, NB, batch, 0)
        plsc.subcore_barrier()

        pltpu.sync_copy(acc.at[pl.ds(s * RPT, RPT)],
                        out.at[c, pl.ds(s * RPT, RPT)])
        if with_count:
            pltpu.sync_copy(cnt_acc.at[pl.ds(s * CPT, CPT)],
                            cnt_out.at[c, pl.ds(s * CPT, CPT)])

    return pl.kernel(body, out_type=out_type, mesh=mesh,
                     scratch_types=scratch,
                     compiler_params=pltpu.CompilerParams(
                         use_tc_tiling_on_sc=False),
                     name=f"sc_seg_sum_d{D}" + ("_cnt" if with_count else ""))


HD = F_IN // 2
_seg_sum_cnt_64 = _make_seg_sum(HD, True)
_seg_sum_64 = _make_seg_sum(HD, False)
_seg_sum_40 = _make_seg_sum(C, False)


BR = 1000


def _tc_a_body(aggpa, aggpb, cntp, x, w1l, w1r, b1, w2l, w2r, b2,
               p2_o, r2b_o, invc_o, dinv_o):
    agg = jnp.concatenate([aggpa[0] + aggpa[1], aggpb[0] + aggpb[1]], axis=1)
    cnt = cntp[0, 0] + cntp[0, 1]
    invc = (1.0 / jnp.maximum(cnt, 1.0))[:, None]
    dinv = lax.rsqrt(cnt + 1.0)[:, None]
    mean = agg * invc
    h1 = jnp.dot(mean, w1l[...], preferred_element_type=jnp.float32)
    h1 = h1 + jnp.dot(x[...], w1r[...], preferred_element_type=jnp.float32)
    h1 = jnp.maximum(h1 + b1[...], 0.0)
    p2_o[...] = jnp.dot(h1, w2l[...], preferred_element_type=jnp.float32)
    r2b_o[...] = jnp.dot(h1, w2r[...],
                         preferred_element_type=jnp.float32) + b2[...]
    invc_o[...] = invc
    dinv_o[...] = dinv


def _tc_b_body(aggp, invc, r2b, wc, dinv, y_o):
    h2 = (aggp[0] + aggp[1]) * invc[...] + r2b[...]
    xw = jnp.dot(h2, wc[...], preferred_element_type=jnp.float32)
    y_o[...] = xw * dinv[...]


def _tc_c_body(aggp, y, dinv, bc, out_o):
    out_o[...] = dinv[...] * (aggp[0] + aggp[1] + y[...]) + bc[...]


def _row_spec(d):
    return pl.BlockSpec((BR, d), lambda i: (i, 0))


def _part_spec(d):
    return pl.BlockSpec((NC, BR, d), lambda i: (0, i, 0))


def _full_spec(a, b):
    return pl.BlockSpec((a, b), lambda i: (0, 0))


_tc_a = pl.pallas_call(
    _tc_a_body,
    grid=(N // BR,),
    in_specs=[
        _part_spec(HD),
        _part_spec(HD),
        pl.BlockSpec((1, NC, BR), lambda i: (i, 0, 0)),
        _row_spec(F_IN),
        _full_spec(F_IN, H), _full_spec(F_IN, H), _full_spec(1, H),
        _full_spec(H, C), _full_spec(H, C), _full_spec(1, C),
    ],
    out_specs=[_row_spec(C), _row_spec(C), _row_spec(1), _row_spec(1)],
    out_shape=[
        jax.ShapeDtypeStruct((N, C), jnp.float32),
        jax.ShapeDtypeStruct((N, C), jnp.float32),
        jax.ShapeDtypeStruct((N, 1), jnp.float32),
        jax.ShapeDtypeStruct((N, 1), jnp.float32),
    ],
    name="tc_dense_a",
)

_tc_b = pl.pallas_call(
    _tc_b_body,
    grid=(N // BR,),
    in_specs=[_part_spec(C), _row_spec(1), _row_spec(C),
              _full_spec(C, C), _row_spec(1)],
    out_specs=[_row_spec(C)],
    out_shape=[jax.ShapeDtypeStruct((N, C), jnp.float32)],
    name="tc_dense_b",
)

_tc_c = pl.pallas_call(
    _tc_c_body,
    grid=(N // BR,),
    in_specs=[_part_spec(C), _row_spec(C), _row_spec(1), _full_spec(1, C)],
    out_specs=[_row_spec(C)],
    out_shape=[jax.ShapeDtypeStruct((N, C), jnp.float32)],
    name="tc_dense_c",
)


@jax.jit
def kernel(x, edge_index, W1l, W1r, b1, W2l, W2r, b2, Wc, bc):
    src3 = edge_index[0].reshape(NW, NCHUNK, K)
    dst3 = edge_index[1].reshape(NW, NCHUNK, K)
    b1r = b1.reshape(1, H)
    b2r = b2.reshape(1, C)
    bcr = bc.reshape(1, C)

    xa = x[:, :HD]
    xb = x[:, HD:]
    agg1pa, cntp_pad = _seg_sum_cnt_64(xa, src3, dst3)
    (agg1pb,) = _seg_sum_64(xb, src3, dst3)
    cntp = cntp_pad[:, :N].reshape(NC, N // BR, BR).transpose(1, 0, 2)
    p2, r2b, invc, dinv = _tc_a(agg1pa, agg1pb, cntp, x, W1l, W1r, b1r,
                                W2l, W2r, b2r)
    (agg2p,) = _seg_sum_40(p2, src3, dst3)
    (y,) = _tc_b(agg2p, invc, r2b, Wc, dinv)
    (agg3p,) = _seg_sum_40(y, src3, dst3)
    (out,) = _tc_c(agg3p, y, dinv, bcr)
    return out
